# baseline (speedup 1.0000x reference)
import functools

import jax
import jax.numpy as jnp
from jax import lax
from jax.experimental import pallas as pl
from jax.experimental.pallas import tpu as pltpu
from jax.experimental.pallas import tpu_sc as plsc

_LANES = 16
_CHUNK = 128


def _make_sc_kernel(batch, embed, nc, ns):
    num_workers = nc * ns
    b_per_w = batch // num_workers
    n_chunks = b_per_w // _CHUNK
    n_streams = embed * n_chunks
    mesh = plsc.VectorSubcoreMesh(core_axis_name="c", subcore_axis_name="s")

    @functools.partial(
        pl.kernel,
        mesh=mesh,
        out_type=jax.ShapeDtypeStruct((batch,), jnp.float32),
        compiler_params=pltpu.CompilerParams(
            needs_layout_passes=False, use_tc_tiling_on_sc=False),
        scratch_types=[
            pltpu.VMEM((b_per_w,), jnp.int32),
            pltpu.VMEM((b_per_w,), jnp.int32),
            pltpu.VMEM((embed * b_per_w,), jnp.float32),
            pltpu.VMEM((embed * b_per_w,), jnp.float32),
            pltpu.VMEM((b_per_w,), jnp.float32),
            pltpu.VMEM((b_per_w,), jnp.float32),
            pltpu.VMEM((_LANES,), jnp.float32),
            pltpu.VMEM((b_per_w,), jnp.float32),
            pltpu.SemaphoreType.DMA,
            pltpu.SemaphoreType.DMA,
        ],
    )
    def k(uids_r, mids_r, ue_r, me_r, ub_r, mb_r, gb_r, out_r,
          uid_v, mid_v, du, dm, bu, bm, gb_v, out_v, sem_t, sem_b):
        wid = lax.axis_index("c") * ns + lax.axis_index("s")
        base = wid * b_per_w

        pltpu.sync_copy(uids_r.at[pl.ds(base, b_per_w)], uid_v)
        pltpu.sync_copy(mids_r.at[pl.ds(base, b_per_w)], mid_v)
        pltpu.sync_copy(gb_r, gb_v)

        for j in range(n_chunks):
            sl = pl.ds(j * _CHUNK, _CHUNK)
            pltpu.make_async_copy(ub_r.at[uid_v.at[sl]], bu.at[sl], sem_b).start()
            pltpu.make_async_copy(mb_r.at[mid_v.at[sl]], bm.at[sl], sem_b).start()

        def fire(s, carry):
            d = s >> 2
            j = s & (n_chunks - 1)
            sl = pl.ds(j * _CHUNK, _CHUNK)
            dst = pl.ds(s * _CHUNK, _CHUNK)
            pltpu.make_async_copy(ue_r.at[d].at[uid_v.at[sl]], du.at[dst], sem_t).start()
            pltpu.make_async_copy(me_r.at[d].at[mid_v.at[sl]], dm.at[dst], sem_t).start()
            return carry

        lax.fori_loop(0, n_streams, fire, 0)

        def drain(s, carry):
            d = s >> 2
            j = s & (n_chunks - 1)
            sl = pl.ds(j * _CHUNK, _CHUNK)
            dst = pl.ds(s * _CHUNK, _CHUNK)
            pltpu.make_async_copy(ue_r.at[d].at[uid_v.at[sl]], du.at[dst], sem_t).wait()
            pltpu.make_async_copy(me_r.at[d].at[mid_v.at[sl]], dm.at[dst], sem_t).wait()
            return carry

        lax.fori_loop(0, n_streams, drain, 0)
        for j in range(n_chunks):
            sl = pl.ds(j * _CHUNK, _CHUNK)
            pltpu.make_async_copy(ub_r.at[uid_v.at[sl]], bu.at[sl], sem_b).wait()
            pltpu.make_async_copy(mb_r.at[mid_v.at[sl]], bm.at[sl], sem_b).wait()

        gb = gb_v[...]

        def dot(g16, carry):
            off = g16 * _LANES
            acc = bu[pl.ds(off, _LANES)] + bm[pl.ds(off, _LANES)] + gb
            for d in range(embed):
                acc = acc + (du[pl.ds(d * b_per_w + off, _LANES)]
                             * dm[pl.ds(d * b_per_w + off, _LANES)])
            out_v[pl.ds(off, _LANES)] = acc
            return carry

        lax.fori_loop(0, b_per_w // _LANES, dot, 0)

        pltpu.sync_copy(out_v, out_r.at[pl.ds(base, b_per_w)])

    return k


def kernel(user_ids, movie_ids, user_embedding, movie_embedding,
           user_bias, movie_bias, global_bias):
    batch = user_ids.shape[0]
    embed = user_embedding.shape[1]
    info = plsc.get_sparse_core_info()
    nc, ns = info.num_cores, info.num_subcores

    k = _make_sc_kernel(batch, embed, nc, ns)
    gb16 = jnp.broadcast_to(jnp.reshape(global_bias, (1,)),
                            (_LANES,)).astype(jnp.float32)
    return k(user_ids.astype(jnp.int32), movie_ids.astype(jnp.int32),
             user_embedding.T, movie_embedding.T,
             user_bias.reshape(-1), movie_bias.reshape(-1), gb16)


# speedup vs baseline: 3.1836x; 3.1836x over previous
import functools

import jax
import jax.numpy as jnp
from jax import lax
from jax.experimental import pallas as pl
from jax.experimental.pallas import tpu as pltpu
from jax.experimental.pallas import tpu_sc as plsc

_LANES = 16
_CHUNK = 128


def _make_sc_kernel(batch, embed_half, num_workers, nc, ns):
    b_per_w = batch // num_workers
    n_chunks = b_per_w // _CHUNK
    n_groups = b_per_w // _LANES
    mesh = plsc.VectorSubcoreMesh(core_axis_name="c", subcore_axis_name="s")

    @functools.partial(
        pl.kernel,
        mesh=mesh,
        out_type=jax.ShapeDtypeStruct((batch,), jnp.float32),
        compiler_params=pltpu.CompilerParams(
            needs_layout_passes=False, use_tc_tiling_on_sc=False),
        scratch_types=[
            pltpu.VMEM((n_chunks, _CHUNK), jnp.int32),
            pltpu.VMEM((n_chunks, _CHUNK), jnp.int32),
            pltpu.VMEM((b_per_w, embed_half), jnp.float32),
            pltpu.VMEM((b_per_w, embed_half), jnp.float32),
            pltpu.VMEM((b_per_w, embed_half), jnp.float32),
            pltpu.VMEM((b_per_w, embed_half), jnp.float32),
            pltpu.VMEM((b_per_w,), jnp.float32),
            pltpu.VMEM((b_per_w,), jnp.float32),
            pltpu.VMEM((_LANES,), jnp.float32),
            pltpu.VMEM((b_per_w,), jnp.float32),
            pltpu.SemaphoreType.DMA,
        ],
    )
    def k(uids_r, mids_r, ua_r, ub2_r, ma_r, mb2_r, ub_r, mb_r, gb_r, out_r,
          idx_u, idx_m, rows_ua, rows_ub, rows_ma, rows_mb,
          bu_v, bm_v, gb_v, out_v, sem):
        wid = lax.axis_index("c") * ns + lax.axis_index("s")

        pltpu.sync_copy(uids_r.at[pl.ds(wid * n_chunks, n_chunks)], idx_u)
        pltpu.sync_copy(mids_r.at[pl.ds(wid * n_chunks, n_chunks)], idx_m)
        pltpu.sync_copy(gb_r, gb_v)

        descs = []
        for j in range(n_chunks):
            sl = pl.ds(j * _CHUNK, _CHUNK)
            descs.append(pltpu.async_copy(ua_r.at[idx_u.at[j]], rows_ua.at[sl], sem))
            descs.append(pltpu.async_copy(ub2_r.at[idx_u.at[j]], rows_ub.at[sl], sem))
            descs.append(pltpu.async_copy(ma_r.at[idx_m.at[j]], rows_ma.at[sl], sem))
            descs.append(pltpu.async_copy(mb2_r.at[idx_m.at[j]], rows_mb.at[sl], sem))
            descs.append(pltpu.async_copy(ub_r.at[idx_u.at[j]], bu_v.at[sl], sem))
            descs.append(pltpu.async_copy(mb_r.at[idx_m.at[j]], bm_v.at[sl], sem))
        for d in descs:
            d.wait()

        gb = gb_v[...]

        def group(g, carry):
            rid = lax.iota(jnp.int32, _LANES) + g * _LANES
            acc = (plsc.load_gather(bu_v, [rid])
                   + plsc.load_gather(bm_v, [rid]) + gb)
            for d in range(embed_half):
                col = jnp.full((_LANES,), d, jnp.int32)
                acc = acc + (plsc.load_gather(rows_ua, [rid, col])
                             * plsc.load_gather(rows_ma, [rid, col]))
                acc = acc + (plsc.load_gather(rows_ub, [rid, col])
                             * plsc.load_gather(rows_mb, [rid, col]))
            plsc.store_scatter(out_v, [rid], acc)
            return carry

        lax.fori_loop(0, n_groups, group, 0)
        pltpu.sync_copy(out_v, out_r.at[pl.ds(wid * b_per_w, b_per_w)])

    return k


def kernel(user_ids, movie_ids, user_embedding, movie_embedding,
           user_bias, movie_bias, global_bias):
    batch = user_ids.shape[0]
    embed = user_embedding.shape[1]
    half = embed // 2
    info = plsc.get_sparse_core_info()
    nc, ns = info.num_cores, info.num_subcores
    num_workers = nc * ns

    k = _make_sc_kernel(batch, half, num_workers, nc, ns)
    uids2 = user_ids.astype(jnp.int32).reshape(batch // _CHUNK, _CHUNK)
    mids2 = movie_ids.astype(jnp.int32).reshape(batch // _CHUNK, _CHUNK)
    gb16 = jnp.broadcast_to(jnp.reshape(global_bias, (1,)),
                            (_LANES,)).astype(jnp.float32)
    return k(uids2, mids2,
             user_embedding[:, :half], user_embedding[:, half:],
             movie_embedding[:, :half], movie_embedding[:, half:],
             user_bias.reshape(-1), movie_bias.reshape(-1), gb16)
